# Initial kernel scaffold; baseline (speedup 1.0000x reference)
import jax
import jax.numpy as jnp
from jax.experimental import pallas as pl


def kernel(user_ids, table):
    raise NotImplementedError("write your pallas kernel here")


# speedup vs baseline: 2.9978x; 2.9978x over previous
import functools

import jax
import jax.numpy as jnp
from jax import lax
from jax.experimental import pallas as pl
from jax.experimental.pallas import tpu as pltpu
from jax.experimental.pallas import tpu_sc as plsc


def _sc_geometry():
    try:
        info = plsc.get_sparse_core_info()
        return info.num_cores, info.num_subcores
    except Exception:
        return 2, 16


@functools.lru_cache(maxsize=None)
def _make_gather(B, V, D, chunk):
    NC, NS = _sc_geometry()
    NW = NC * NS
    assert B % (NW * chunk) == 0
    steps = B // (NW * chunk)
    b_per_w = B // NW
    mesh = plsc.VectorSubcoreMesh(core_axis_name="c", subcore_axis_name="s",
                                  num_cores=NC, num_subcores=NS)

    @functools.partial(
        pl.kernel,
        out_type=jax.ShapeDtypeStruct((B, D), jnp.float32),
        mesh=mesh,
        scratch_types=[
            pltpu.VMEM((chunk,), jnp.int32),
            pltpu.VMEM((chunk, D), jnp.float32),
            pltpu.SemaphoreType.DMA,
        ],
        compiler_params=pltpu.CompilerParams(use_tc_tiling_on_sc=False),
    )
    def gather_kernel(idx_hbm, table_hbm, out_hbm, idx_v, rows_v, sem):
        wid = lax.axis_index("s") * NC + lax.axis_index("c")
        base = wid * b_per_w

        def body(i, _):
            off = base + i * chunk
            pltpu.sync_copy(idx_hbm.at[pl.ds(off, chunk)], idx_v)
            pltpu.async_copy(table_hbm.at[idx_v], rows_v, sem).wait()
            pltpu.sync_copy(rows_v, out_hbm.at[pl.ds(off, chunk)])
            return _

        lax.fori_loop(0, steps, body, None)

    return gather_kernel


def kernel(user_ids, table):
    Bv, S = user_ids.shape
    V, D = table.shape
    B = Bv * S
    flat_idx = user_ids.reshape(B).astype(jnp.int32)
    out = _make_gather(B, V, D, 2560)(flat_idx, table)
    return out.reshape(Bv, S, D)


# speedup vs baseline: 3.0130x; 1.0051x over previous
import functools

import jax
import jax.numpy as jnp
from jax import lax
from jax.experimental import pallas as pl
from jax.experimental.pallas import tpu as pltpu
from jax.experimental.pallas import tpu_sc as plsc


def _sc_geometry():
    try:
        info = plsc.get_sparse_core_info()
        return info.num_cores, info.num_subcores
    except Exception:
        return 2, 16


@functools.lru_cache(maxsize=None)
def _make_gather(B, V, D, chunk):
    NC, NS = _sc_geometry()
    NW = NC * NS
    assert B % (NW * 2 * chunk) == 0
    b_per_w = B // NW
    steps = b_per_w // chunk
    pairs = steps // 2
    mesh = plsc.VectorSubcoreMesh(core_axis_name="c", subcore_axis_name="s",
                                  num_cores=NC, num_subcores=NS)

    @functools.partial(
        pl.kernel,
        out_type=jax.ShapeDtypeStruct((B, D), jnp.float32),
        mesh=mesh,
        scratch_types=[
            pltpu.VMEM((b_per_w,), jnp.int32),
            pltpu.VMEM((chunk, D), jnp.float32),
            pltpu.VMEM((chunk, D), jnp.float32),
            pltpu.SemaphoreType.DMA,
            pltpu.SemaphoreType.DMA,
            pltpu.SemaphoreType.DMA,
            pltpu.SemaphoreType.DMA,
        ],
        compiler_params=pltpu.CompilerParams(use_tc_tiling_on_sc=False),
    )
    def gather_kernel(idx_hbm, table_hbm, out_hbm, idx_v, rows0, rows1,
                      sg0, sg1, sw0, sw1):
        wid = lax.axis_index("s") * NC + lax.axis_index("c")
        base = wid * b_per_w
        rows_v = (rows0, rows1)
        sg = (sg0, sg1)
        sw = (sw0, sw1)

        pltpu.sync_copy(idx_hbm.at[pl.ds(base, b_per_w)], idx_v)

        def gather_copy(i, b):
            return pltpu.make_async_copy(
                table_hbm.at[idx_v.at[pl.ds(i * chunk, chunk)]],
                rows_v[b], sg[b])

        def write_copy(i, b):
            return pltpu.make_async_copy(
                rows_v[b], out_hbm.at[pl.ds(base + i * chunk, chunk)], sw[b])

        gather_copy(0, 0).start()

        def body(p, _):
            i = 2 * p

            @pl.when(p > 0)
            def _():
                write_copy(i - 1, 1).wait()

            gather_copy(i + 1, 1).start()
            gather_copy(i, 0).wait()
            write_copy(i, 0).start()

            @pl.when(p < pairs - 1)
            def _():
                write_copy(i, 0).wait()
                gather_copy(i + 2, 0).start()

            gather_copy(i + 1, 1).wait()
            write_copy(i + 1, 1).start()
            return _

        lax.fori_loop(0, pairs, body, None)
        write_copy(steps - 2, 0).wait()
        write_copy(steps - 1, 1).wait()

    return gather_kernel


def kernel(user_ids, table):
    Bv, S = user_ids.shape
    V, D = table.shape
    B = Bv * S
    flat_idx = user_ids.reshape(B).astype(jnp.int32)
    out = _make_gather(B, V, D, 1280)(flat_idx, table)
    return out.reshape(Bv, S, D)


# speedup vs baseline: 5.9370x; 1.9705x over previous
import functools

import jax
import jax.numpy as jnp
from jax import lax
from jax.experimental import pallas as pl
from jax.experimental.pallas import tpu as pltpu
from jax.experimental.pallas import tpu_sc as plsc


def _sc_geometry():
    try:
        info = plsc.get_sparse_core_info()
        return info.num_cores, info.num_subcores
    except Exception:
        return 2, 16


@functools.lru_cache(maxsize=None)
def _make_gather_t(S, B, V, D, chunk):
    NC, NS = _sc_geometry()
    NW = NC * NS
    assert D == NW and B % chunk == 0 and chunk % 16 == 0
    n_chunks = B // chunk
    mesh = plsc.VectorSubcoreMesh(core_axis_name="c", subcore_axis_name="s",
                                  num_cores=NC, num_subcores=NS)

    @functools.partial(
        pl.kernel,
        out_type=jax.ShapeDtypeStruct((S, D, B), jnp.float32),
        mesh=mesh,
        scratch_types=[
            pltpu.VMEM((V,), jnp.float32),
            pltpu.VMEM((chunk,), jnp.int32),
            pltpu.VMEM((chunk,), jnp.float32),
        ],
        compiler_params=pltpu.CompilerParams(use_tc_tiling_on_sc=True,
                                             needs_layout_passes=False),
    )
    def gather_kernel(uid_hbm, table_hbm, out_hbm, row_v, idx_v, res_v):
        d = lax.axis_index("s") * NC + lax.axis_index("c")
        pltpu.sync_copy(table_hbm.at[d, :], row_v)

        def chunk_body(t, _):
            s = t // n_chunks
            b0 = (t % n_chunks) * chunk
            pltpu.sync_copy(uid_hbm.at[s, pl.ds(b0, chunk)], idx_v)

            def inner(j, _):
                iv = idx_v[pl.ds(j * 16, 16)]
                res_v[pl.ds(j * 16, 16)] = plsc.load_gather(row_v, [iv])
                return _

            lax.fori_loop(0, chunk // 16, inner, None, unroll=8)
            pltpu.sync_copy(res_v, out_hbm.at[s, d, pl.ds(b0, chunk)])
            return _

        lax.fori_loop(0, S * n_chunks, chunk_body, None)

    return gather_kernel


def kernel(user_ids, table):
    Bv, S = user_ids.shape
    V, D = table.shape
    uid_t = user_ids.T.astype(jnp.int32)
    table_t = table.T
    out_t = _make_gather_t(S, Bv, V, D, 8192)(uid_t, table_t)
    return out_t.transpose(2, 0, 1)


# speedup vs baseline: 14.1336x; 2.3806x over previous
import functools

import jax
import jax.numpy as jnp
from jax import lax
from jax.experimental import pallas as pl
from jax.experimental.pallas import tpu as pltpu
from jax.experimental.pallas import tpu_sc as plsc


def _sc_geometry():
    try:
        info = plsc.get_sparse_core_info()
        return info.num_cores, info.num_subcores
    except Exception:
        return 2, 16


@functools.lru_cache(maxsize=None)
def _make_gather_t(S, B, V, D, chunk):
    NC, NS = _sc_geometry()
    NW = NC * NS
    assert D == NW and B % chunk == 0 and chunk % 16 == 0
    n_chunks = B // chunk
    mesh = plsc.VectorSubcoreMesh(core_axis_name="c", subcore_axis_name="s",
                                  num_cores=NC, num_subcores=NS)

    @functools.partial(
        pl.kernel,
        out_type=jax.ShapeDtypeStruct((S, D, B), jnp.float32),
        mesh=mesh,
        scratch_types=[
            pltpu.VMEM((V,), jnp.float32),
            pltpu.VMEM((chunk,), jnp.int32),
            pltpu.VMEM((chunk,), jnp.float32),
        ],
        compiler_params=pltpu.CompilerParams(use_tc_tiling_on_sc=True,
                                             needs_layout_passes=False,
                                             disable_bounds_checks=True),
    )
    def gather_kernel(uid_hbm, table_hbm, out_hbm, row_v, idx_v, res_v):
        d = lax.axis_index("s") * NC + lax.axis_index("c")
        pltpu.sync_copy(table_hbm.at[d, :], row_v)

        def chunk_body(t, _):
            s = t // n_chunks
            b0 = (t % n_chunks) * chunk
            pltpu.sync_copy(uid_hbm.at[s, pl.ds(b0, chunk)], idx_v)

            @plsc.parallel_loop(0, chunk, step=16, unroll=8)
            def _inner(off):
                iv = idx_v[pl.ds(off, 16)]
                res_v[pl.ds(off, 16)] = plsc.load_gather(row_v, [iv])
            pltpu.sync_copy(res_v, out_hbm.at[s, d, pl.ds(b0, chunk)])
            return _

        lax.fori_loop(0, S * n_chunks, chunk_body, None)

    return gather_kernel


def kernel(user_ids, table):
    Bv, S = user_ids.shape
    V, D = table.shape
    uid_t = user_ids.T.astype(jnp.int32)
    table_t = table.T
    out_t = _make_gather_t(S, Bv, V, D, 8192)(uid_t, table_t)
    return out_t.transpose(2, 0, 1)


# speedup vs baseline: 15.8872x; 1.1241x over previous
import functools

import jax
import jax.numpy as jnp
from jax import lax
from jax.experimental import pallas as pl
from jax.experimental.pallas import tpu as pltpu
from jax.experimental.pallas import tpu_sc as plsc


def _sc_geometry():
    try:
        info = plsc.get_sparse_core_info()
        return info.num_cores, info.num_subcores
    except Exception:
        return 2, 16


@functools.lru_cache(maxsize=None)
def _make_gather_t(S, B, V, D, chunk):
    NC, NS = _sc_geometry()
    NW = NC * NS
    assert D == NW and B % chunk == 0 and chunk % 16 == 0
    n_chunks = B // chunk
    n_tiles = S * n_chunks
    assert n_tiles % 2 == 0
    pairs = n_tiles // 2
    mesh = plsc.VectorSubcoreMesh(core_axis_name="c", subcore_axis_name="s",
                                  num_cores=NC, num_subcores=NS)

    @functools.partial(
        pl.kernel,
        out_type=jax.ShapeDtypeStruct((S, D, B), jnp.float32),
        mesh=mesh,
        scratch_types=[
            pltpu.VMEM((V,), jnp.float32),
            pltpu.VMEM((chunk,), jnp.int32),
            pltpu.VMEM((chunk,), jnp.int32),
            pltpu.VMEM((chunk,), jnp.float32),
            pltpu.VMEM((chunk,), jnp.float32),
            pltpu.SemaphoreType.DMA,
            pltpu.SemaphoreType.DMA,
            pltpu.SemaphoreType.DMA,
            pltpu.SemaphoreType.DMA,
        ],
        compiler_params=pltpu.CompilerParams(use_tc_tiling_on_sc=True,
                                             needs_layout_passes=False,
                                             disable_bounds_checks=True),
    )
    def gather_kernel(uid_hbm, table_hbm, out_hbm, row_v, idx0, idx1,
                      res0, res1, si0, si1, so0, so1):
        d = lax.axis_index("s") * NC + lax.axis_index("c")
        pltpu.sync_copy(table_hbm.at[d, :], row_v)
        idx_v = (idx0, idx1)
        res_v = (res0, res1)
        si = (si0, si1)
        so = (so0, so1)

        def idx_copy(t, b):
            s = t // n_chunks
            b0 = (t % n_chunks) * chunk
            return pltpu.make_async_copy(
                uid_hbm.at[s, pl.ds(b0, chunk)], idx_v[b], si[b])

        def out_copy(t, b):
            s = t // n_chunks
            b0 = (t % n_chunks) * chunk
            return pltpu.make_async_copy(
                res_v[b], out_hbm.at[s, d, pl.ds(b0, chunk)], so[b])

        def compute(b):
            @plsc.parallel_loop(0, chunk, step=16, unroll=8)
            def _inner(off):
                iv = idx_v[b][pl.ds(off, 16)]
                res_v[b][pl.ds(off, 16)] = plsc.load_gather(row_v, [iv])

        idx_copy(0, 0).start()

        def body(p, _):
            t = 2 * p
            idx_copy(t, 0).wait()

            @pl.when(p > 0)
            def _():
                out_copy(t - 1, 1).wait()

            idx_copy(t + 1, 1).start()

            @pl.when(p > 0)
            def _():
                out_copy(t - 2, 0).wait()

            compute(0)
            out_copy(t, 0).start()
            idx_copy(t + 1, 1).wait()

            @pl.when(p < pairs - 1)
            def _():
                idx_copy(t + 2, 0).start()

            compute(1)
            out_copy(t + 1, 1).start()
            return _

        lax.fori_loop(0, pairs, body, None)
        out_copy(n_tiles - 2, 0).wait()
        out_copy(n_tiles - 1, 1).wait()

    return gather_kernel


def kernel(user_ids, table):
    Bv, S = user_ids.shape
    V, D = table.shape
    uid_t = user_ids.T.astype(jnp.int32)
    table_t = table.T
    out_t = _make_gather_t(S, Bv, V, D, 4096)(uid_t, table_t)
    return out_t.transpose(2, 0, 1)
